# baseline (speedup 1.0000x reference)
import functools

import jax
import jax.numpy as jnp
from jax import lax
from jax.experimental import pallas as pl
from jax.experimental.pallas import tpu as pltpu
from jax.experimental.pallas import tpu_sc as plsc

_TMR = 256
_L = 16


def _gating_kernel(x_ref, wg_ref, e01_ref, g01_ref, *, n_experts):
    xf = x_ref[...]
    logits = jnp.dot(xf, wg_ref[...], preferred_element_type=jnp.float32)
    col = lax.broadcasted_iota(jnp.int32, logits.shape, 1)
    l1 = jnp.max(logits, axis=1, keepdims=True)
    a1 = jnp.min(jnp.where(logits >= l1, col, n_experts), axis=1, keepdims=True)
    l2m = jnp.where(col == a1, -jnp.inf, logits)
    l2 = jnp.max(l2m, axis=1, keepdims=True)
    a2 = jnp.min(jnp.where(l2m >= l2, col, n_experts), axis=1, keepdims=True)
    z = jnp.exp(l2 - l1)
    den = 1.0 + z
    tm = a1.shape[0]
    e01_ref[...] = jnp.concatenate(
        [a1.reshape(1, tm), a2.reshape(1, tm)], axis=0)
    g01_ref[...] = jnp.concatenate(
        [(1.0 / den).reshape(1, tm), (z / den).reshape(1, tm)], axis=0)


def _gating(x, w_gate):
    n, d = x.shape
    e = w_gate.shape[1]
    tm = 1024
    return pl.pallas_call(
        functools.partial(_gating_kernel, n_experts=e),
        grid=(n // tm,),
        in_specs=[
            pl.BlockSpec((tm, d), lambda t: (t, 0)),
            pl.BlockSpec((d, e), lambda t: (0, 0)),
        ],
        out_specs=[
            pl.BlockSpec((2, tm), lambda t: (0, t)),
            pl.BlockSpec((2, tm), lambda t: (0, t)),
        ],
        out_shape=[
            jax.ShapeDtypeStruct((2, n), jnp.int32),
            jax.ShapeDtypeStruct((2, n), jnp.float32),
        ],
        compiler_params=pltpu.CompilerParams(
            dimension_semantics=("parallel",),
        ),
    )(x, w_gate)


def _make_routing(n_assign, n_pad, n_experts, n_tiles_pad):
    per_w = n_assign // _L
    zero_per_w = n_pad // _L
    mesh = plsc.VectorSubcoreMesh(core_axis_name="c", subcore_axis_name="s",
                                  num_cores=1)

    @functools.partial(
        pl.kernel,
        out_type=[
            jax.ShapeDtypeStruct((n_pad,), jnp.int32),
            jax.ShapeDtypeStruct((n_pad,), jnp.float32),
            jax.ShapeDtypeStruct((n_assign,), jnp.int32),
            jax.ShapeDtypeStruct((n_tiles_pad,), jnp.int32),
        ],
        mesh=mesh,
        scratch_types=[
            pltpu.VMEM((per_w,), jnp.int32),
            pltpu.VMEM((per_w,), jnp.float32),
            pltpu.VMEM((per_w,), jnp.int32),
            pltpu.VMEM((per_w,), jnp.int32),
            pltpu.VMEM((zero_per_w,), jnp.int32),
            pltpu.VMEM((zero_per_w,), jnp.float32),
            pltpu.VMEM((_L,), jnp.int32),
            pltpu.VMEM((_L * _L,), jnp.int32),
            pltpu.VMEM((n_tiles_pad,), jnp.int32),
            pltpu.VMEM_SHARED((_L * _L,), jnp.int32),
            pltpu.SemaphoreType.DMA,
            pltpu.SemaphoreType.DMA,
        ],
        compiler_params=pltpu.CompilerParams(needs_layout_passes=False),
    )
    def routing(ids_hbm, gts_hbm, gtok_hbm, ggate_hbm, pos_hbm, te_hbm,
                ids_v, gts_v, pos_v, tok_v, ztok_v, zgate_v, hist_v,
                table_v, te_v, shared_h, sem0, sem1):
        w = lax.axis_index("s")
        base = w * per_w
        lane = lax.iota(jnp.int32, _L)
        zeros = jnp.zeros((_L,), jnp.int32)

        pltpu.sync_copy(ids_hbm.at[pl.ds(base, per_w)], ids_v)
        pltpu.sync_copy(gts_hbm.at[pl.ds(base, per_w)], gts_v)

        for j in range(zero_per_w // _L):
            ztok_v[pl.ds(j * _L, _L)] = zeros
            zgate_v[pl.ds(j * _L, _L)] = jnp.zeros((_L,), jnp.float32)
        pltpu.sync_copy(ztok_v, gtok_hbm.at[pl.ds(w * zero_per_w, zero_per_w)])
        pltpu.sync_copy(zgate_v, ggate_hbm.at[pl.ds(w * zero_per_w, zero_per_w)])

        ones = jnp.ones((_L,), jnp.int32)

        def _splat(s):
            return jnp.full((_L,), s, jnp.int32)

        def hist_body(j, acc):
            v = ids_v[pl.ds(j * _L, _L)]
            for ei in range(n_experts):
                m = jnp.where(v == _splat(ei), ones, zeros)
                c = jnp.sum(m)
                acc = acc + jnp.where(lane == _splat(ei), _splat(c), zeros)
            return acc

        hist = lax.fori_loop(0, per_w // _L, hist_body, zeros)
        hist_v[...] = hist
        pltpu.sync_copy(hist_v, shared_h.at[pl.ds(w * _L, _L)])
        plsc.subcore_barrier()
        pltpu.sync_copy(shared_h, table_v)

        tot = zeros
        mypre = zeros
        for wp in range(_L):
            row = table_v[pl.ds(wp * _L, _L)]
            tot = tot + row
            sel = jnp.where(w > wp, 1, 0)
            mypre = mypre + row * _splat(sel)
        c255 = _splat(_TMR - 1)
        padded = ((tot + c255) // _splat(_TMR)) * _splat(_TMR)
        base_pad = plsc.cumsum(padded) - padded
        starts = base_pad + mypre

        start_e = []
        for ei in range(n_experts):
            s = jnp.sum(jnp.where(lane == _splat(ei), starts, zeros))
            start_e.append(_splat(s))

        def place_body(j, carry):
            se = list(carry)
            v = ids_v[pl.ds(j * _L, _L)]
            pos = jnp.zeros((_L,), jnp.int32)
            for ei in range(n_experts):
                m = v == _splat(ei)
                mi = jnp.where(m, ones, zeros)
                r = plsc.cumsum(mi)
                pos = jnp.where(m, se[ei] + r - ones, pos)
                se[ei] = se[ei] + _splat(jnp.sum(mi))
            pos_v[pl.ds(j * _L, _L)] = pos
            tok_v[pl.ds(j * _L, _L)] = (
                (_splat(base + j * _L) + lane) & _splat(n_assign // 2 - 1))
            return tuple(se)

        lax.fori_loop(0, per_w // _L, place_body, tuple(start_e))

        pltpu.sync_copy(pos_v, pos_hbm.at[pl.ds(base, per_w)])
        plsc.subcore_barrier()
        pltpu.async_copy(tok_v, gtok_hbm.at[pos_v], sem0).wait()
        pltpu.async_copy(gts_v, ggate_hbm.at[pos_v], sem1).wait()

        @pl.when(w == 0)
        def _():
            bpd = base_pad // _splat(_TMR)
            bpd_e = []
            for ei in range(n_experts):
                s = jnp.sum(jnp.where(lane == _splat(ei), bpd, zeros))
                bpd_e.append(_splat(s))
            for jb in range(n_tiles_pad // _L):
                ntv = lane + _splat(jb * _L)
                te = jnp.full((_L,), -1, jnp.int32)
                for ei in range(n_experts):
                    te = te + jnp.where(ntv >= bpd_e[ei], ones, zeros)
                te_v[pl.ds(jb * _L, _L)] = te
            pltpu.sync_copy(te_v, te_hbm)

    return routing


def _make_gather(n_pad, d):
    mesh = plsc.VectorSubcoreMesh(core_axis_name="c", subcore_axis_name="s")
    n_workers = 32
    per_w = n_pad // n_workers
    chunk = 24
    n_chunks = per_w // chunk
    assert per_w % chunk == 0 and n_chunks % 2 == 0

    @functools.partial(
        pl.kernel,
        out_type=jax.ShapeDtypeStruct((n_pad, d), jnp.float32),
        mesh=mesh,
        scratch_types=[
            pltpu.VMEM((per_w,), jnp.int32),
            pltpu.VMEM((chunk, d), jnp.float32),
            pltpu.VMEM((chunk, d), jnp.float32),
            pltpu.SemaphoreType.DMA,
            pltpu.SemaphoreType.DMA,
        ],
        compiler_params=pltpu.CompilerParams(needs_layout_passes=False),
    )
    def gather(x_hbm, gtok_hbm, xg_hbm, idx_v, rows0_v, rows1_v, sem0, sem1):
        wid = lax.axis_index("s") * 2 + lax.axis_index("c")
        base = wid * per_w
        rows = (rows0_v, rows1_v)
        sems = (sem0, sem1)

        pltpu.sync_copy(gtok_hbm.at[pl.ds(base, per_w)], idx_v)

        def start(c, b):
            pltpu.make_async_copy(x_hbm.at[idx_v.at[pl.ds(c * chunk, chunk)]],
                                  rows[b], sems[b]).start()

        start(0, 0)

        def body(jj, _):
            for b in range(2):
                c = jj * 2 + b

                @pl.when(c + 1 < n_chunks)
                def _():
                    start(c + 1, 1 - b)

                pltpu.make_async_copy(
                    x_hbm.at[idx_v.at[pl.ds(c * chunk, chunk)]],
                    rows[b], sems[b]).wait()
                pltpu.sync_copy(rows[b], xg_hbm.at[pl.ds(base + c * chunk,
                                                         chunk)])
            return 0

        lax.fori_loop(0, n_chunks // 2, body, 0)

    return gather


_NT = (((1,), (1,)), ((), ()))


def _gmm_kernel(te_ref, xg_ref, w_ref, b_ref, gg_ref, y_ref, wb_ref, last_ref):
    t = pl.program_id(0)
    te_cur = te_ref[t]

    @pl.when((t == 0) | (te_cur != last_ref[0]))
    def _():
        wb_ref[...] = w_ref[0].astype(jnp.bfloat16)
        last_ref[0] = te_cur

    wb = wb_ref[...]
    xb = xg_ref[...].astype(jnp.bfloat16)
    y = lax.dot_general(xb, wb, _NT, preferred_element_type=jnp.float32)
    corr = lax.dot_general(b_ref[0].astype(jnp.bfloat16), wb, _NT,
                           preferred_element_type=jnp.float32)
    y_ref[...] = gg_ref[...] * (y - corr)


def _grouped_matmul(te, xg, w, bias3, gg2d, n_tiles):
    n_pad, d = xg.shape
    o = w.shape[1]
    grid_spec = pltpu.PrefetchScalarGridSpec(
        num_scalar_prefetch=1,
        grid=(n_tiles,),
        in_specs=[
            pl.BlockSpec((_TMR, d), lambda t, te_r: (t, 0)),
            pl.BlockSpec((1, o, d), lambda t, te_r: (te_r[t], 0, 0)),
            pl.BlockSpec((1, 1, d), lambda t, te_r: (te_r[t], 0, 0)),
            pl.BlockSpec((_TMR, 1), lambda t, te_r: (t, 0)),
        ],
        out_specs=pl.BlockSpec((_TMR, o), lambda t, te_r: (t, 0)),
        scratch_shapes=[
            pltpu.VMEM((o, d), jnp.bfloat16),
            pltpu.SMEM((1,), jnp.int32),
        ],
    )
    return pl.pallas_call(
        _gmm_kernel,
        grid_spec=grid_spec,
        out_shape=jax.ShapeDtypeStruct((n_pad, o), jnp.float32),
        compiler_params=pltpu.CompilerParams(
            dimension_semantics=("arbitrary",),
        ),
    )(te, xg, w, bias3, gg2d)


def _make_combine(n, n_pad, o):
    mesh = plsc.VectorSubcoreMesh(core_axis_name="c", subcore_axis_name="s")
    n_workers = 32
    per_w = n // n_workers
    chunk = 8
    n_chunks = per_w // chunk

    @functools.partial(
        pl.kernel,
        out_type=jax.ShapeDtypeStruct((n, o), jnp.float32),
        mesh=mesh,
        scratch_types=[
            pltpu.VMEM((per_w,), jnp.int32),
            pltpu.VMEM((per_w,), jnp.int32),
            pltpu.VMEM((chunk, o), jnp.float32),
            pltpu.VMEM((chunk, o), jnp.float32),
            pltpu.VMEM((chunk, o), jnp.float32),
            pltpu.VMEM((chunk, o), jnp.float32),
            pltpu.SemaphoreType.DMA,
            pltpu.SemaphoreType.DMA,
            pltpu.SemaphoreType.DMA,
            pltpu.SemaphoreType.DMA,
        ],
        compiler_params=pltpu.CompilerParams(needs_layout_passes=False),
    )
    def combine(yflat_hbm, pos0_hbm, pos1_hbm, out_hbm,
                i0_v, i1_v, bufa0, bufb0, bufa1, bufb1,
                sa0, sb0, sa1, sb1):
        wid = lax.axis_index("s") * 2 + lax.axis_index("c")
        base_tok = wid * per_w
        bufa = (bufa0, bufa1)
        bufb = (bufb0, bufb1)
        sa = (sa0, sa1)
        sb = (sb0, sb1)

        pltpu.sync_copy(pos0_hbm.at[pl.ds(base_tok, per_w)], i0_v)
        pltpu.sync_copy(pos1_hbm.at[pl.ds(base_tok, per_w)], i1_v)

        def start(c, b):
            sl = pl.ds(c * chunk, chunk)
            pltpu.make_async_copy(yflat_hbm.at[i0_v.at[sl]], bufa[b],
                                  sa[b]).start()
            pltpu.make_async_copy(yflat_hbm.at[i1_v.at[sl]], bufb[b],
                                  sb[b]).start()

        start(0, 0)

        def body(jj, _):
            for b in range(2):
                c = jj * 2 + b

                @pl.when(c + 1 < n_chunks)
                def _():
                    start(c + 1, 1 - b)

                sl = pl.ds(c * chunk, chunk)
                pltpu.make_async_copy(yflat_hbm.at[i0_v.at[sl]], bufa[b],
                                      sa[b]).wait()
                pltpu.make_async_copy(yflat_hbm.at[i1_v.at[sl]], bufb[b],
                                      sb[b]).wait()
                for r in range(chunk):
                    def add_body(k, _2):
                        for u in range(4):
                            slu = pl.ds(k * 4 * _L + u * _L, _L)
                            bufa[b][r, slu] = bufa[b][r, slu] + bufb[b][r, slu]
                        return 0
                    lax.fori_loop(0, o // (4 * _L), add_body, 0)
                pltpu.sync_copy(bufa[b],
                                out_hbm.at[pl.ds(base_tok + c * chunk, chunk)])
            return 0

        lax.fori_loop(0, n_chunks // 2, body, 0)

    return combine


def kernel(x, w_gate, w_noise, expert_bias, expert_weight):
    del w_noise
    n, d = x.shape
    e = w_gate.shape[1]
    o = expert_weight.shape[1]
    k = 2
    n_assign = n * k
    n_tiles = n_assign // _TMR + e
    n_pad = n_tiles * _TMR
    n_tiles_pad = ((n_tiles + _L - 1) // _L) * _L

    e01, g01 = _gating(x, w_gate)
    ids_flat = e01.reshape(n_assign)
    gts_flat = g01.reshape(n_assign)

    routing = _make_routing(n_assign, n_pad, e, n_tiles_pad)
    gtok, ggate, pos01, te = routing(ids_flat, gts_flat)

    gather = _make_gather(n_pad, d)
    xg = gather(x, gtok)

    bias3 = expert_bias.reshape(e, 1, d)
    gg2d = ggate.reshape(n_pad, 1)
    yflat = _grouped_matmul(te, xg, expert_weight, bias3, gg2d, n_tiles)

    combine = _make_combine(n, n_pad, o)
    out = combine(yflat, pos01[:n], pos01[n:])

    load_loss = jnp.asarray(0.0, dtype=jnp.float32)
    return (out, load_loss)


# speedup vs baseline: 1.0109x; 1.0109x over previous
import functools

import jax
import jax.numpy as jnp
from jax import lax
from jax.experimental import pallas as pl
from jax.experimental.pallas import tpu as pltpu
from jax.experimental.pallas import tpu_sc as plsc

_TMR = 256
_L = 16


def _gating_kernel(x_ref, wg_ref, e01_ref, g01_ref, *, n_experts):
    xf = x_ref[...]
    logits = jnp.dot(xf, wg_ref[...], preferred_element_type=jnp.float32)
    col = lax.broadcasted_iota(jnp.int32, logits.shape, 1)
    l1 = jnp.max(logits, axis=1, keepdims=True)
    a1 = jnp.min(jnp.where(logits >= l1, col, n_experts), axis=1, keepdims=True)
    l2m = jnp.where(col == a1, -jnp.inf, logits)
    l2 = jnp.max(l2m, axis=1, keepdims=True)
    a2 = jnp.min(jnp.where(l2m >= l2, col, n_experts), axis=1, keepdims=True)
    z = jnp.exp(l2 - l1)
    den = 1.0 + z
    tm = a1.shape[0]
    e01_ref[...] = jnp.concatenate(
        [a1.reshape(1, tm), a2.reshape(1, tm)], axis=0)
    g01_ref[...] = jnp.concatenate(
        [(1.0 / den).reshape(1, tm), (z / den).reshape(1, tm)], axis=0)


def _gating(x, w_gate):
    n, d = x.shape
    e = w_gate.shape[1]
    tm = 1024
    return pl.pallas_call(
        functools.partial(_gating_kernel, n_experts=e),
        grid=(n // tm,),
        in_specs=[
            pl.BlockSpec((tm, d), lambda t: (t, 0)),
            pl.BlockSpec((d, e), lambda t: (0, 0)),
        ],
        out_specs=[
            pl.BlockSpec((2, tm), lambda t: (0, t)),
            pl.BlockSpec((2, tm), lambda t: (0, t)),
        ],
        out_shape=[
            jax.ShapeDtypeStruct((2, n), jnp.int32),
            jax.ShapeDtypeStruct((2, n), jnp.float32),
        ],
        compiler_params=pltpu.CompilerParams(
            dimension_semantics=("parallel",),
        ),
    )(x, w_gate)


def _make_routing(n_assign, n_pad, n_experts, n_tiles_pad):
    per_w = n_assign // _L
    zero_per_w = n_pad // _L
    mesh = plsc.VectorSubcoreMesh(core_axis_name="c", subcore_axis_name="s",
                                  num_cores=1)

    @functools.partial(
        pl.kernel,
        out_type=[
            jax.ShapeDtypeStruct((n_pad,), jnp.int32),
            jax.ShapeDtypeStruct((n_pad,), jnp.float32),
            jax.ShapeDtypeStruct((n_assign,), jnp.int32),
            jax.ShapeDtypeStruct((n_tiles_pad,), jnp.int32),
        ],
        mesh=mesh,
        scratch_types=[
            pltpu.VMEM((per_w,), jnp.int32),
            pltpu.VMEM((per_w,), jnp.float32),
            pltpu.VMEM((per_w,), jnp.int32),
            pltpu.VMEM((per_w,), jnp.int32),
            pltpu.VMEM((zero_per_w,), jnp.int32),
            pltpu.VMEM((zero_per_w,), jnp.float32),
            pltpu.VMEM((_L,), jnp.int32),
            pltpu.VMEM((_L * _L,), jnp.int32),
            pltpu.VMEM((n_tiles_pad,), jnp.int32),
            pltpu.VMEM_SHARED((_L * _L,), jnp.int32),
            pltpu.SemaphoreType.DMA,
            pltpu.SemaphoreType.DMA,
        ],
        compiler_params=pltpu.CompilerParams(needs_layout_passes=False),
    )
    def routing(ids_hbm, gts_hbm, gtok_hbm, ggate_hbm, pos_hbm, te_hbm,
                ids_v, gts_v, pos_v, tok_v, ztok_v, zgate_v, hist_v,
                table_v, te_v, shared_h, sem0, sem1):
        w = lax.axis_index("s")
        base = w * per_w
        lane = lax.iota(jnp.int32, _L)
        zeros = jnp.zeros((_L,), jnp.int32)

        pltpu.sync_copy(ids_hbm.at[pl.ds(base, per_w)], ids_v)
        pltpu.sync_copy(gts_hbm.at[pl.ds(base, per_w)], gts_v)

        for j in range(zero_per_w // _L):
            ztok_v[pl.ds(j * _L, _L)] = zeros
            zgate_v[pl.ds(j * _L, _L)] = jnp.zeros((_L,), jnp.float32)
        pltpu.sync_copy(ztok_v, gtok_hbm.at[pl.ds(w * zero_per_w, zero_per_w)])
        pltpu.sync_copy(zgate_v, ggate_hbm.at[pl.ds(w * zero_per_w, zero_per_w)])

        ones = jnp.ones((_L,), jnp.int32)

        def _splat(s):
            return jnp.full((_L,), s, jnp.int32)

        def hist_body(j, acc):
            v = ids_v[pl.ds(j * _L, _L)]
            for ei in range(n_experts):
                m = jnp.where(v == _splat(ei), ones, zeros)
                c = jnp.sum(m)
                acc = acc + jnp.where(lane == _splat(ei), _splat(c), zeros)
            return acc

        hist = lax.fori_loop(0, per_w // _L, hist_body, zeros)
        hist_v[...] = hist
        pltpu.sync_copy(hist_v, shared_h.at[pl.ds(w * _L, _L)])
        plsc.subcore_barrier()
        pltpu.sync_copy(shared_h, table_v)

        tot = zeros
        mypre = zeros
        for wp in range(_L):
            row = table_v[pl.ds(wp * _L, _L)]
            tot = tot + row
            sel = jnp.where(w > wp, 1, 0)
            mypre = mypre + row * _splat(sel)
        c255 = _splat(_TMR - 1)
        padded = ((tot + c255) // _splat(_TMR)) * _splat(_TMR)
        base_pad = plsc.cumsum(padded) - padded
        starts = base_pad + mypre

        start_e = []
        for ei in range(n_experts):
            s = jnp.sum(jnp.where(lane == _splat(ei), starts, zeros))
            start_e.append(_splat(s))

        def place_body(j, carry):
            se = list(carry)
            v = ids_v[pl.ds(j * _L, _L)]
            pos = jnp.zeros((_L,), jnp.int32)
            for ei in range(n_experts):
                m = v == _splat(ei)
                mi = jnp.where(m, ones, zeros)
                r = plsc.cumsum(mi)
                pos = jnp.where(m, se[ei] + r - ones, pos)
                se[ei] = se[ei] + _splat(jnp.sum(mi))
            pos_v[pl.ds(j * _L, _L)] = pos
            tok_v[pl.ds(j * _L, _L)] = (
                (_splat(base + j * _L) + lane) & _splat(n_assign // 2 - 1))
            return tuple(se)

        lax.fori_loop(0, per_w // _L, place_body, tuple(start_e))

        pltpu.sync_copy(pos_v, pos_hbm.at[pl.ds(base, per_w)])
        plsc.subcore_barrier()
        pltpu.async_copy(tok_v, gtok_hbm.at[pos_v], sem0).wait()
        pltpu.async_copy(gts_v, ggate_hbm.at[pos_v], sem1).wait()

        @pl.when(w == 0)
        def _():
            bpd = base_pad // _splat(_TMR)
            bpd_e = []
            for ei in range(n_experts):
                s = jnp.sum(jnp.where(lane == _splat(ei), bpd, zeros))
                bpd_e.append(_splat(s))
            for jb in range(n_tiles_pad // _L):
                ntv = lane + _splat(jb * _L)
                te = jnp.full((_L,), -1, jnp.int32)
                for ei in range(n_experts):
                    te = te + jnp.where(ntv >= bpd_e[ei], ones, zeros)
                te_v[pl.ds(jb * _L, _L)] = te
            pltpu.sync_copy(te_v, te_hbm)

    return routing


def _make_gather(n_pad, d):
    mesh = plsc.VectorSubcoreMesh(core_axis_name="c", subcore_axis_name="s")
    n_workers = 32
    per_w = n_pad // n_workers
    chunk = 24
    n_chunks = per_w // chunk
    assert per_w % chunk == 0 and n_chunks % 2 == 0

    @functools.partial(
        pl.kernel,
        out_type=jax.ShapeDtypeStruct((n_pad, d), jnp.float32),
        mesh=mesh,
        scratch_types=[
            pltpu.VMEM((per_w,), jnp.int32),
            pltpu.VMEM((chunk, d), jnp.float32),
            pltpu.VMEM((chunk, d), jnp.float32),
            pltpu.SemaphoreType.DMA,
            pltpu.SemaphoreType.DMA,
        ],
        compiler_params=pltpu.CompilerParams(needs_layout_passes=False),
    )
    def gather(x_hbm, gtok_hbm, xg_hbm, idx_v, rows0_v, rows1_v, sem0, sem1):
        wid = lax.axis_index("s") * 2 + lax.axis_index("c")
        base = wid * per_w
        rows = (rows0_v, rows1_v)
        sems = (sem0, sem1)

        pltpu.sync_copy(gtok_hbm.at[pl.ds(base, per_w)], idx_v)

        def start(c, b):
            pltpu.make_async_copy(x_hbm.at[idx_v.at[pl.ds(c * chunk, chunk)]],
                                  rows[b], sems[b]).start()

        start(0, 0)

        def body(jj, _):
            for b in range(2):
                c = jj * 2 + b

                @pl.when(c + 1 < n_chunks)
                def _():
                    start(c + 1, 1 - b)

                pltpu.make_async_copy(
                    x_hbm.at[idx_v.at[pl.ds(c * chunk, chunk)]],
                    rows[b], sems[b]).wait()
                pltpu.sync_copy(rows[b], xg_hbm.at[pl.ds(base + c * chunk,
                                                         chunk)])
            return 0

        lax.fori_loop(0, n_chunks // 2, body, 0)

    return gather


_NT = (((1,), (1,)), ((), ()))


def _gmm_kernel(te_ref, xg_ref, w_ref, b_ref, gg_ref, y_ref):
    del te_ref
    wb = w_ref[0].astype(jnp.bfloat16)
    xb = xg_ref[...].astype(jnp.bfloat16)
    y = lax.dot_general(xb, wb, _NT, preferred_element_type=jnp.float32)
    corr = lax.dot_general(b_ref[0].astype(jnp.bfloat16), wb, _NT,
                           preferred_element_type=jnp.float32)
    y_ref[...] = gg_ref[...] * (y - corr)


def _grouped_matmul(te, xg, w, bias3, gg2d, n_tiles):
    n_pad, d = xg.shape
    o = w.shape[1]
    grid_spec = pltpu.PrefetchScalarGridSpec(
        num_scalar_prefetch=1,
        grid=(n_tiles,),
        in_specs=[
            pl.BlockSpec((_TMR, d), lambda t, te_r: (t, 0)),
            pl.BlockSpec((1, o, d), lambda t, te_r: (te_r[t], 0, 0)),
            pl.BlockSpec((1, 1, d), lambda t, te_r: (te_r[t], 0, 0)),
            pl.BlockSpec((_TMR, 1), lambda t, te_r: (t, 0)),
        ],
        out_specs=pl.BlockSpec((_TMR, o), lambda t, te_r: (t, 0)),
    )
    return pl.pallas_call(
        _gmm_kernel,
        grid_spec=grid_spec,
        out_shape=jax.ShapeDtypeStruct((n_pad, o), jnp.float32),
        compiler_params=pltpu.CompilerParams(
            dimension_semantics=("arbitrary",),
        ),
    )(te, xg, w, bias3, gg2d)


def _make_combine(n, n_pad, o):
    mesh = plsc.VectorSubcoreMesh(core_axis_name="c", subcore_axis_name="s")
    n_workers = 32
    per_w = n // n_workers
    chunk = 8
    n_chunks = per_w // chunk

    @functools.partial(
        pl.kernel,
        out_type=jax.ShapeDtypeStruct((n, o), jnp.float32),
        mesh=mesh,
        scratch_types=[
            pltpu.VMEM((per_w,), jnp.int32),
            pltpu.VMEM((per_w,), jnp.int32),
            pltpu.VMEM((chunk, o), jnp.float32),
            pltpu.VMEM((chunk, o), jnp.float32),
            pltpu.VMEM((chunk, o), jnp.float32),
            pltpu.VMEM((chunk, o), jnp.float32),
            pltpu.SemaphoreType.DMA,
            pltpu.SemaphoreType.DMA,
            pltpu.SemaphoreType.DMA,
            pltpu.SemaphoreType.DMA,
        ],
        compiler_params=pltpu.CompilerParams(needs_layout_passes=False),
    )
    def combine(yflat_hbm, pos0_hbm, pos1_hbm, out_hbm,
                i0_v, i1_v, bufa0, bufb0, bufa1, bufb1,
                sa0, sb0, sa1, sb1):
        wid = lax.axis_index("s") * 2 + lax.axis_index("c")
        base_tok = wid * per_w
        bufa = (bufa0, bufa1)
        bufb = (bufb0, bufb1)
        sa = (sa0, sa1)
        sb = (sb0, sb1)

        pltpu.sync_copy(pos0_hbm.at[pl.ds(base_tok, per_w)], i0_v)
        pltpu.sync_copy(pos1_hbm.at[pl.ds(base_tok, per_w)], i1_v)

        def start(c, b):
            sl = pl.ds(c * chunk, chunk)
            pltpu.make_async_copy(yflat_hbm.at[i0_v.at[sl]], bufa[b],
                                  sa[b]).start()
            pltpu.make_async_copy(yflat_hbm.at[i1_v.at[sl]], bufb[b],
                                  sb[b]).start()

        start(0, 0)

        def body(jj, _):
            for b in range(2):
                c = jj * 2 + b

                @pl.when(c + 1 < n_chunks)
                def _():
                    start(c + 1, 1 - b)

                sl = pl.ds(c * chunk, chunk)
                pltpu.make_async_copy(yflat_hbm.at[i0_v.at[sl]], bufa[b],
                                      sa[b]).wait()
                pltpu.make_async_copy(yflat_hbm.at[i1_v.at[sl]], bufb[b],
                                      sb[b]).wait()
                for r in range(chunk):
                    def add_body(k, _2):
                        for u in range(4):
                            slu = pl.ds(k * 4 * _L + u * _L, _L)
                            bufa[b][r, slu] = bufa[b][r, slu] + bufb[b][r, slu]
                        return 0
                    lax.fori_loop(0, o // (4 * _L), add_body, 0)
                pltpu.sync_copy(bufa[b],
                                out_hbm.at[pl.ds(base_tok + c * chunk, chunk)])
            return 0

        lax.fori_loop(0, n_chunks // 2, body, 0)

    return combine


def kernel(x, w_gate, w_noise, expert_bias, expert_weight):
    del w_noise
    n, d = x.shape
    e = w_gate.shape[1]
    o = expert_weight.shape[1]
    k = 2
    n_assign = n * k
    n_tiles = n_assign // _TMR + e
    n_pad = n_tiles * _TMR
    n_tiles_pad = ((n_tiles + _L - 1) // _L) * _L

    e01, g01 = _gating(x, w_gate)
    ids_flat = e01.reshape(n_assign)
    gts_flat = g01.reshape(n_assign)

    routing = _make_routing(n_assign, n_pad, e, n_tiles_pad)
    gtok, ggate, pos01, te = routing(ids_flat, gts_flat)

    gather = _make_gather(n_pad, d)
    xg = gather(x, gtok)

    bias3 = expert_bias.reshape(e, 1, d)
    gg2d = ggate.reshape(n_pad, 1)
    yflat = _grouped_matmul(te, xg, expert_weight, bias3, gg2d, n_tiles)

    combine = _make_combine(n, n_pad, o)
    out = combine(yflat, pos01[:n], pos01[n:])

    load_loss = jnp.asarray(0.0, dtype=jnp.float32)
    return (out, load_loss)
